# baseline (speedup 1.0000x reference)
import functools

import jax
import jax.numpy as jnp
from jax import lax
from jax.experimental import pallas as pl
from jax.experimental.pallas import tpu as pltpu
from jax.experimental.pallas import tpu_sc as plsc

NC = 2
NS = 16
NW = NC * NS


def _round_up(x, m):
    return (x + m - 1) // m * m


@functools.cache
def _make_sc_agg(n_nodes, d, e, k, with_deg=True):
    assert e % NW == 0
    ew = e // NW
    assert ew % k == 0 and k % 8 == 0 and k <= 128
    nwin = ew // k
    ch = min(nwin, 40)
    assert nwin % ch == 0 and (ch % 8 == 0 or nwin == ch)
    nchunks = nwin // ch
    assert n_nodes % (NS * 8) == 0
    rows = n_nodes // NS
    ndeg = _round_up(n_nodes, 128)
    degw = ndeg // NS

    mesh = plsc.VectorSubcoreMesh(
        core_axis_name="c", subcore_axis_name="s", num_cores=NC,
        num_subcores=NS)

    @functools.partial(
        pl.kernel,
        mesh=mesh,
        out_type=(
            jax.ShapeDtypeStruct((NC * n_nodes, d), jnp.float32),
            jax.ShapeDtypeStruct((NC * ndeg,), jnp.float32),
        ),
        scratch_types=[
            pltpu.VMEM((ch, k), jnp.int32),
            pltpu.VMEM((ch, k), jnp.int32),
            pltpu.VMEM((2, k, d), jnp.float32),
            pltpu.VMEM((k,), jnp.float32),
            pltpu.VMEM((degw,), jnp.float32),
            pltpu.VMEM_SHARED((n_nodes, d), jnp.float32),
            pltpu.VMEM_SHARED((ndeg,), jnp.float32),
            pltpu.SemaphoreType.DMA((2,)),
            pltpu.SemaphoreType.DMA((2,)),
            pltpu.SemaphoreType.DMA((2,)),
        ],
    )
    def sc_agg(table, src, dst, zrows, zdeg, ones,
               agg_out, deg_out,
               sidx, didx, rowbuf, ones_v, deg_v, agg_sh, deg_sh,
               gsem, ssem, dsem):
        c = lax.axis_index("c")
        s = lax.axis_index("s")
        w = c * NS + s
        pltpu.sync_copy(zrows, agg_sh.at[pl.ds(s * rows, rows)])
        pltpu.sync_copy(zdeg, deg_v)
        pltpu.sync_copy(deg_v, deg_sh.at[pl.ds(s * degw, degw)])
        pltpu.sync_copy(ones, ones_v)
        plsc.subcore_barrier()

        def start_gather(j, b):
            return pltpu.async_copy(table.at[sidx.at[j]], rowbuf.at[b],
                                    gsem.at[b])

        for ci in range(nchunks):
            pltpu.sync_copy(src.at[w, pl.ds(ci * ch, ch)], sidx)
            pltpu.sync_copy(dst.at[w, pl.ds(ci * ch, ch)], didx)
            start_gather(0, 0)

            def body(j, carry):
                b = lax.rem(j, 2)
                nb = 1 - b

                @pl.when(j >= 1)
                def _():
                    pltpu.make_async_copy(rowbuf.at[nb],
                                          agg_sh.at[didx.at[j]],
                                          ssem.at[nb]).wait()
                    if with_deg:
                        pltpu.make_async_copy(ones_v, deg_sh.at[didx.at[j]],
                                              dsem.at[nb]).wait()

                @pl.when(j + 1 < ch)
                def _():
                    start_gather(j + 1, nb)

                pltpu.make_async_copy(table.at[sidx.at[j]], rowbuf.at[b],
                                      gsem.at[b]).wait()
                pltpu.async_copy(rowbuf.at[b], agg_sh.at[didx.at[j]],
                                 ssem.at[b], add=True)
                if with_deg:
                    pltpu.async_copy(ones_v, deg_sh.at[didx.at[j]],
                                     dsem.at[b], add=True)
                return carry

            lax.fori_loop(0, ch, body, 0)
            bl = (ch - 1) % 2
            pltpu.make_async_copy(rowbuf.at[bl], agg_sh.at[didx.at[0]],
                                  ssem.at[bl]).wait()
            if with_deg:
                pltpu.make_async_copy(ones_v, deg_sh.at[didx.at[0]],
                                      dsem.at[bl]).wait()
        plsc.subcore_barrier()
        pltpu.sync_copy(agg_sh.at[pl.ds(s * rows, rows)],
                        agg_out.at[pl.ds(c * n_nodes + s * rows, rows)])
        pltpu.sync_copy(deg_sh.at[pl.ds(s * degw, degw)], deg_v)
        pltpu.sync_copy(deg_v, deg_out.at[pl.ds(c * ndeg + s * degw, degw)])

    def run(table, src, dst):
        zrows = jnp.zeros((rows, d), jnp.float32)
        zdeg = jnp.zeros((degw,), jnp.float32)
        ones = jnp.ones((k,), jnp.float32)
        src3 = src.reshape(NW, nwin, k)
        dst3 = dst.reshape(NW, nwin, k)
        agg, deg = sc_agg(table, src3, dst3, zrows, zdeg, ones)
        agg = agg.reshape(NC, n_nodes, d)
        deg = deg.reshape(NC, ndeg)
        return agg, deg

    return run


def _tc_matmul(x, w):
    n, din = x.shape
    dout = w.shape[1]
    blk = 1000

    def body(x_ref, w_ref, o_ref):
        o_ref[...] = jnp.dot(x_ref[...], w_ref[...],
                             preferred_element_type=jnp.float32)

    return pl.pallas_call(
        body,
        grid=(n // blk,),
        in_specs=[
            pl.BlockSpec((blk, din), lambda i: (i, 0)),
            pl.BlockSpec((din, dout), lambda i: (0, 0)),
        ],
        out_specs=pl.BlockSpec((blk, dout), lambda i: (i, 0)),
        out_shape=jax.ShapeDtypeStruct((n, dout), jnp.float32),
    )(x, w)


def _tc_norm_act_mm(a0, a1, d0, d1, b, w, relu):
    n, din = a0.shape
    blk = 1024 if n % 1024 == 0 else 1000

    def body(a0_ref, a1_ref, d0_ref, d1_ref, b_ref, *rest):
        if w is not None:
            w_ref, o_ref = rest
        else:
            (o_ref,) = rest
        deg = jnp.maximum(d0_ref[...] + d1_ref[...], 1.0)
        h = (a0_ref[...] + a1_ref[...]) / deg + b_ref[...]
        if relu:
            h = jnp.maximum(h, 0.0)
        if w is not None:
            h = jnp.dot(h, w_ref[...], preferred_element_type=jnp.float32)
        o_ref[...] = h

    dout = din if w is None else w.shape[1]
    in_specs = [
        pl.BlockSpec((blk, din), lambda i: (i, 0)),
        pl.BlockSpec((blk, din), lambda i: (i, 0)),
        pl.BlockSpec((blk, 1), lambda i: (i, 0)),
        pl.BlockSpec((blk, 1), lambda i: (i, 0)),
        pl.BlockSpec((1, din), lambda i: (0, 0)),
    ]
    args = [a0, a1, d0.reshape(n, 1), d1.reshape(n, 1), b.reshape(1, din)]
    if w is not None:
        in_specs.append(pl.BlockSpec((din, dout), lambda i: (0, 0)))
        args.append(w)
    return pl.pallas_call(
        body,
        grid=(n // blk,),
        in_specs=in_specs,
        out_specs=pl.BlockSpec((blk, dout), lambda i: (i, 0)),
        out_shape=jax.ShapeDtypeStruct((n, dout), jnp.float32),
    )(*args)


N_INNER = 10000
N_OUTER = 1000
E_INNER = 320000
E_OUTER = 16000
NPAD_INNER = 10240
NPAD_OUTER = 1024
K = 128
EPAD_INNER = 327680
EPAD_OUTER = 16384
EPAD_POOL = 12288


def _pad_edges(src, dst, e_pad, n_dummy_base, n_dummy):
    pad = e_pad - src.shape[0]
    nsrc = jnp.arange(pad, dtype=jnp.int32) % jnp.int32(997)
    ndst = n_dummy_base + jnp.arange(pad, dtype=jnp.int32) % jnp.int32(n_dummy)
    return (jnp.concatenate([src.astype(jnp.int32), nsrc]),
            jnp.concatenate([dst.astype(jnp.int32), ndst]))


def kernel(inner_x, inner_edge_index, inner_graph_ids, outer_edge_index,
           W_in0, b_in0, W_in1, b_in1, W_out0, b_out0, W_out1, b_out1):
    src, dst = _pad_edges(inner_edge_index[0], inner_edge_index[1],
                          EPAD_INNER, N_INNER, NPAD_INNER - N_INNER)
    y0 = _tc_matmul(inner_x, W_in0)
    agg_inner = _make_sc_agg(NPAD_INNER, 128, EPAD_INNER, K)
    a, dg = agg_inner(y0, src, dst)
    y1 = _tc_norm_act_mm(a[0], a[1], dg[0], dg[1], b_in0, W_in1, relu=True)

    agg_inner2 = _make_sc_agg(NPAD_INNER, 128, EPAD_INNER, K, with_deg=False)
    a2, _ = agg_inner2(y1, src, dst)
    h2 = _tc_norm_act_mm(a2[0], a2[1], dg[0], dg[1], b_in1, None, relu=False)

    psrc, pdst = _pad_edges(jnp.arange(N_INNER, dtype=jnp.int32),
                            inner_graph_ids, EPAD_POOL,
                            N_OUTER, 24)
    agg_pool = _make_sc_agg(NPAD_OUTER, 128, EPAD_POOL, K)
    p, cnt = agg_pool(h2, psrc, pdst)
    zero_b = jnp.zeros((128,), jnp.float32)
    feats = _tc_norm_act_mm(p[0, :N_OUTER], p[1, :N_OUTER],
                            cnt[0, :N_OUTER], cnt[1, :N_OUTER],
                            zero_b, W_out0, relu=False)

    osrc, odst = _pad_edges(outer_edge_index[0], outer_edge_index[1],
                            EPAD_OUTER, N_OUTER, 24)
    agg_outer = _make_sc_agg(NPAD_OUTER, 128, EPAD_OUTER, K)
    o, odg = agg_outer(feats, osrc, odst)
    w_out1p = jnp.pad(W_out1, ((0, 0), (0, 128 - W_out1.shape[1])))
    u1 = _tc_norm_act_mm(o[0, :N_OUTER], o[1, :N_OUTER],
                         odg[0, :N_OUTER], odg[1, :N_OUTER],
                         b_out0, w_out1p, relu=True)

    agg_outer2 = _make_sc_agg(NPAD_OUTER, 128, EPAD_OUTER, K, with_deg=False)
    q, _ = agg_outer2(u1, osrc, odst)
    out = _tc_norm_act_mm(q[0, :N_OUTER, :64], q[1, :N_OUTER, :64],
                          odg[0, :N_OUTER], odg[1, :N_OUTER],
                          b_out1, None, relu=False)
    return out


# speedup vs baseline: 1.0141x; 1.0141x over previous
import functools

import jax
import jax.numpy as jnp
from jax import lax
from jax.experimental import pallas as pl
from jax.experimental.pallas import tpu as pltpu
from jax.experimental.pallas import tpu_sc as plsc

NC = 2
NS = 16
NW = NC * NS


def _round_up(x, m):
    return (x + m - 1) // m * m


@functools.cache
def _make_sc_agg(n_nodes, d, e, k, with_deg=True):
    assert e % NW == 0
    ew = e // NW
    assert ew % k == 0 and k % 8 == 0 and k <= 128
    nwin = ew // k
    ch = min(nwin, 40)
    assert nwin % ch == 0 and (ch % 8 == 0 or nwin == ch)
    nchunks = nwin // ch
    assert n_nodes % (NS * 8) == 0
    rows = n_nodes // NS
    ndeg = _round_up(n_nodes, 128)
    degw = ndeg // NS

    mesh = plsc.VectorSubcoreMesh(
        core_axis_name="c", subcore_axis_name="s", num_cores=NC,
        num_subcores=NS)

    @functools.partial(
        pl.kernel,
        mesh=mesh,
        out_type=(
            jax.ShapeDtypeStruct((NC * n_nodes, d), jnp.float32),
            jax.ShapeDtypeStruct((NC * ndeg,), jnp.float32),
        ),
        scratch_types=[
            pltpu.VMEM((ch, k), jnp.int32),
            pltpu.VMEM((ch, k), jnp.int32),
            pltpu.VMEM((2, k, d), jnp.float32),
            pltpu.VMEM((k,), jnp.float32),
            pltpu.VMEM((degw,), jnp.float32),
            pltpu.VMEM_SHARED((n_nodes, d), jnp.float32),
            pltpu.VMEM_SHARED((ndeg,), jnp.float32),
            pltpu.SemaphoreType.DMA((2,)),
            pltpu.SemaphoreType.DMA((2,)),
            pltpu.SemaphoreType.DMA((2,)),
        ],
    )
    def sc_agg(table, src, dst, zrows, zdeg, ones,
               agg_out, deg_out,
               sidx, didx, rowbuf, ones_v, deg_v, agg_sh, deg_sh,
               gsem, ssem, dsem):
        c = lax.axis_index("c")
        s = lax.axis_index("s")
        w = c * NS + s
        pltpu.sync_copy(zrows, agg_sh.at[pl.ds(s * rows, rows)])
        pltpu.sync_copy(zdeg, deg_v)
        pltpu.sync_copy(deg_v, deg_sh.at[pl.ds(s * degw, degw)])
        pltpu.sync_copy(ones, ones_v)
        plsc.subcore_barrier()

        def start_gather(j, b):
            return pltpu.async_copy(table.at[sidx.at[j]], rowbuf.at[b],
                                    gsem.at[b])

        for ci in range(nchunks):
            pltpu.sync_copy(src.at[w, pl.ds(ci * ch, ch)], sidx)
            pltpu.sync_copy(dst.at[w, pl.ds(ci * ch, ch)], didx)
            start_gather(0, 0)

            def body(j, carry):
                b = lax.rem(j, 2)
                nb = 1 - b

                @pl.when(j >= 1)
                def _():
                    pltpu.make_async_copy(rowbuf.at[nb],
                                          agg_sh.at[didx.at[j]],
                                          ssem.at[nb]).wait()
                    if with_deg:
                        pltpu.make_async_copy(ones_v, deg_sh.at[didx.at[j]],
                                              dsem.at[nb]).wait()

                @pl.when(j + 1 < ch)
                def _():
                    start_gather(j + 1, nb)

                pltpu.make_async_copy(table.at[sidx.at[j]], rowbuf.at[b],
                                      gsem.at[b]).wait()
                pltpu.async_copy(rowbuf.at[b], agg_sh.at[didx.at[j]],
                                 ssem.at[b], add=True)
                if with_deg:
                    pltpu.async_copy(ones_v, deg_sh.at[didx.at[j]],
                                     dsem.at[b], add=True)
                return carry

            lax.fori_loop(0, ch, body, 0)
            bl = (ch - 1) % 2
            pltpu.make_async_copy(rowbuf.at[bl], agg_sh.at[didx.at[0]],
                                  ssem.at[bl]).wait()
            if with_deg:
                pltpu.make_async_copy(ones_v, deg_sh.at[didx.at[0]],
                                      dsem.at[bl]).wait()
        plsc.subcore_barrier()
        pltpu.sync_copy(agg_sh.at[pl.ds(s * rows, rows)],
                        agg_out.at[pl.ds(c * n_nodes + s * rows, rows)])
        pltpu.sync_copy(deg_sh.at[pl.ds(s * degw, degw)], deg_v)
        pltpu.sync_copy(deg_v, deg_out.at[pl.ds(c * ndeg + s * degw, degw)])

    def run(table, src, dst):
        zrows = jnp.zeros((rows, d), jnp.float32)
        zdeg = jnp.zeros((degw,), jnp.float32)
        ones = jnp.ones((k,), jnp.float32)
        src3 = src.reshape(NW, nwin, k)
        dst3 = dst.reshape(NW, nwin, k)
        agg, deg = sc_agg(table, src3, dst3, zrows, zdeg, ones)
        agg = agg.reshape(NC, n_nodes, d)
        deg = deg.reshape(NC, ndeg)
        return agg, deg

    return run


def _tc_matmul(x, w):
    n, din = x.shape
    dout = w.shape[1]
    blk = 1000

    def body(x_ref, w_ref, o_ref):
        o_ref[...] = jnp.dot(x_ref[...], w_ref[...],
                             preferred_element_type=jnp.float32)

    return pl.pallas_call(
        body,
        grid=(n // blk,),
        in_specs=[
            pl.BlockSpec((blk, din), lambda i: (i, 0)),
            pl.BlockSpec((din, dout), lambda i: (0, 0)),
        ],
        out_specs=pl.BlockSpec((blk, dout), lambda i: (i, 0)),
        out_shape=jax.ShapeDtypeStruct((n, dout), jnp.float32),
    )(x, w)


def _tc_norm_act_mm(a0, a1, d0, d1, b, w, relu):
    n, din = a0.shape
    blk = 1024 if n % 1024 == 0 else 1000

    def body(a0_ref, a1_ref, d0_ref, d1_ref, b_ref, *rest):
        if w is not None:
            w_ref, o_ref = rest
        else:
            (o_ref,) = rest
        deg = jnp.maximum(d0_ref[...] + d1_ref[...], 1.0)
        h = (a0_ref[...] + a1_ref[...]) / deg + b_ref[...]
        if relu:
            h = jnp.maximum(h, 0.0)
        if w is not None:
            h = jnp.dot(h, w_ref[...], preferred_element_type=jnp.float32)
        o_ref[...] = h

    dout = din if w is None else w.shape[1]
    in_specs = [
        pl.BlockSpec((blk, din), lambda i: (i, 0)),
        pl.BlockSpec((blk, din), lambda i: (i, 0)),
        pl.BlockSpec((blk, 1), lambda i: (i, 0)),
        pl.BlockSpec((blk, 1), lambda i: (i, 0)),
        pl.BlockSpec((1, din), lambda i: (0, 0)),
    ]
    args = [a0, a1, d0.reshape(n, 1), d1.reshape(n, 1), b.reshape(1, din)]
    if w is not None:
        in_specs.append(pl.BlockSpec((din, dout), lambda i: (0, 0)))
        args.append(w)
    return pl.pallas_call(
        body,
        grid=(n // blk,),
        in_specs=in_specs,
        out_specs=pl.BlockSpec((blk, dout), lambda i: (i, 0)),
        out_shape=jax.ShapeDtypeStruct((n, dout), jnp.float32),
    )(*args)


N_INNER = 10000
N_OUTER = 1000
E_INNER = 320000
E_OUTER = 16000
NPAD_INNER = 10240
NPAD_OUTER = 1024
K = 128
EPAD_INNER = 327680
EPAD_OUTER = 16384
EPAD_POOL = 12288


def _pad_edges(src, dst, e_pad, n_dummy_base, n_dummy):
    pad = e_pad - src.shape[0]
    nsrc = jnp.arange(pad, dtype=jnp.int32) % jnp.int32(997)
    ndst = n_dummy_base + jnp.arange(pad, dtype=jnp.int32) % jnp.int32(n_dummy)
    return (jnp.concatenate([src.astype(jnp.int32), nsrc]),
            jnp.concatenate([dst.astype(jnp.int32), ndst]))


def kernel(inner_x, inner_edge_index, inner_graph_ids, outer_edge_index,
           W_in0, b_in0, W_in1, b_in1, W_out0, b_out0, W_out1, b_out1):
    src, dst = _pad_edges(inner_edge_index[0], inner_edge_index[1],
                          EPAD_INNER, N_INNER, NPAD_INNER - N_INNER)
    y0 = _tc_matmul(inner_x, W_in0)
    agg_inner = _make_sc_agg(NPAD_INNER, 128, EPAD_INNER, K)
    a, dg = agg_inner(y0, src, dst)
    a, dg = a[:, :N_INNER], dg[:, :N_INNER]
    y1 = _tc_norm_act_mm(a[0], a[1], dg[0], dg[1], b_in0, W_in1, relu=True)

    agg_inner2 = _make_sc_agg(NPAD_INNER, 128, EPAD_INNER, K, with_deg=False)
    a2, _ = agg_inner2(y1, src, dst)
    a2 = a2[:, :N_INNER]
    h2 = _tc_norm_act_mm(a2[0], a2[1], dg[0], dg[1], b_in1, None, relu=False)

    psrc, pdst = _pad_edges(jnp.arange(N_INNER, dtype=jnp.int32),
                            inner_graph_ids, EPAD_POOL,
                            N_OUTER, 24)
    agg_pool = _make_sc_agg(NPAD_OUTER, 128, EPAD_POOL, K)
    p, cnt = agg_pool(h2, psrc, pdst)
    zero_b = jnp.zeros((128,), jnp.float32)
    feats = _tc_norm_act_mm(p[0, :N_OUTER], p[1, :N_OUTER],
                            cnt[0, :N_OUTER], cnt[1, :N_OUTER],
                            zero_b, W_out0, relu=False)

    osrc, odst = _pad_edges(outer_edge_index[0], outer_edge_index[1],
                            EPAD_OUTER, N_OUTER, 24)
    agg_outer = _make_sc_agg(NPAD_OUTER, 128, EPAD_OUTER, K)
    o, odg = agg_outer(feats, osrc, odst)
    w_out1p = jnp.pad(W_out1, ((0, 0), (0, 128 - W_out1.shape[1])))
    u1 = _tc_norm_act_mm(o[0, :N_OUTER], o[1, :N_OUTER],
                         odg[0, :N_OUTER], odg[1, :N_OUTER],
                         b_out0, w_out1p, relu=True)

    agg_outer2 = _make_sc_agg(NPAD_OUTER, 128, EPAD_OUTER, K, with_deg=False)
    q, _ = agg_outer2(u1, osrc, odst)
    out = _tc_norm_act_mm(q[0, :N_OUTER, :64], q[1, :N_OUTER, :64],
                          odg[0, :N_OUTER], odg[1, :N_OUTER],
                          b_out1, None, relu=False)
    return out
